# baseline (speedup 1.0000x reference)
import functools

import jax
import jax.numpy as jnp
from jax import lax
from jax.experimental import pallas as pl
from jax.experimental.pallas import tpu as pltpu
from jax.experimental.pallas import tpu_sc as plsc

N_NODES = 10000
D = 128

NC = 2
NS = 16
NW = NC * NS

CHUNK = 128
ROWS_PER_TILE = 640
N_PAD = NS * ROWS_PER_TILE


NBUF = 1


def _sc_agg_body(feat, src_hbm, dst_hbm, agg_out,
                 src_v, dst_v, rows_v,
                 sg0, agg_s):
  c = lax.axis_index("c")
  s = lax.axis_index("s")
  w = c * NS + s
  n_chunks = src_hbm.shape[1]
  sg = [sg0]

  pltpu.sync_copy(src_hbm.at[w], src_v)
  pltpu.sync_copy(dst_hbm.at[w], dst_v)
  def zrow(i, carry):
    for j in range(D // 16):
      rows_v[0, i, pl.ds(j * 16, 16)] = jnp.zeros((16,), jnp.float32)
    return carry
  lax.fori_loop(0, CHUNK, zrow, 0)
  for zb in range(ROWS_PER_TILE // CHUNK):
    pltpu.sync_copy(rows_v.at[0],
                    agg_s.at[pl.ds(s * ROWS_PER_TILE + zb * CHUNK, CHUNK)])
  plsc.subcore_barrier()

  for b in range(NBUF):
    pltpu.async_copy(feat.at[src_v.at[b]], rows_v.at[b], sg[b])

  def round_body(r, carry):
    cg = r * NBUF
    for b in range(NBUF):
      ci = cg + b
      pltpu.make_async_copy(feat.at[src_v.at[ci]], rows_v.at[b], sg[b]).wait()
      pltpu.sync_copy(rows_v.at[b], agg_s.at[dst_v.at[ci]], add=True)

      nci = ci + NBUF

      @pl.when(nci < n_chunks)
      def _():
        pltpu.async_copy(feat.at[src_v.at[nci]], rows_v.at[b], sg[b])
    return carry

  lax.fori_loop(0, n_chunks // NBUF, round_body, 0)
  plsc.subcore_barrier()

  for blk in range(ROWS_PER_TILE // CHUNK):
    r0 = s * ROWS_PER_TILE + blk * CHUNK
    pltpu.sync_copy(agg_s.at[pl.ds(r0, CHUNK)], rows_v.at[0])
    pltpu.sync_copy(rows_v.at[0], agg_out.at[c, pl.ds(r0, CHUNK)])


def _make_sc_agg(n_chunks):
  mesh = plsc.VectorSubcoreMesh(core_axis_name="c", subcore_axis_name="s")
  out_type = jax.ShapeDtypeStruct((NC, N_PAD, D), jnp.float32)
  scratch = [
      pltpu.VMEM((n_chunks, CHUNK), jnp.int32),
      pltpu.VMEM((n_chunks, CHUNK), jnp.int32),
      pltpu.VMEM((NBUF, CHUNK, D), jnp.float32),
  ] + [pltpu.SemaphoreType.DMA] * NBUF + [
      pltpu.VMEM_SHARED((N_PAD, D), jnp.float32),
  ]
  return pl.kernel(_sc_agg_body, out_type=out_type, mesh=mesh,
                   scratch_types=scratch, name="sc_agg")


def _tc_dense_body(relu, a0, a1, d0, d1, xr, wl, wr, b, o):
  deg = d0[:, 0:1] + d1[:, 0:1]
  inv = 1.0 / jnp.maximum(deg, 1.0)
  mean = (a0[...] + a1[...]) * inv
  acc = (jnp.dot(mean, wl[...], preferred_element_type=jnp.float32)
         + jnp.dot(xr[...], wr[...], preferred_element_type=jnp.float32)
         + b[...])
  o[...] = jnp.maximum(acc, 0.0) if relu else acc


def _make_tc_dense(relu, bn=1000):
  grid = (N_NODES // bn,)
  return pl.pallas_call(
      functools.partial(_tc_dense_body, relu),
      grid=grid,
      in_specs=[
          pl.BlockSpec((bn, D), lambda i: (i, 0)),
          pl.BlockSpec((bn, D), lambda i: (i, 0)),
          pl.BlockSpec((bn, D), lambda i: (i, 0)),
          pl.BlockSpec((bn, D), lambda i: (i, 0)),
          pl.BlockSpec((bn, D), lambda i: (i, 0)),
          pl.BlockSpec((D, D), lambda i: (0, 0)),
          pl.BlockSpec((D, D), lambda i: (0, 0)),
          pl.BlockSpec((1, D), lambda i: (0, 0)),
      ],
      out_specs=pl.BlockSpec((bn, D), lambda i: (i, 0)),
      out_shape=jax.ShapeDtypeStruct((N_NODES, D), jnp.float32),
      name="tc_dense_relu" if relu else "tc_dense",
  )


def kernel(x, edge_index, W1_l, W1_r, b1, W2_l, W2_r, b2):
  e = edge_index.shape[1]
  n_chunks = -(-e // (NW * CHUNK * 8)) * 8
  per_tile = n_chunks * CHUNK
  e_pad = NW * per_tile

  src = edge_index[0].astype(jnp.int32)
  dst = edge_index[1].astype(jnp.int32)
  pad = e_pad - e
  src = jnp.concatenate([src, jnp.zeros((pad,), jnp.int32)]).reshape(NW, n_chunks, CHUNK)
  dst = jnp.concatenate([dst, jnp.full((pad,), N_NODES, jnp.int32)]).reshape(NW, n_chunks, CHUNK)

  ones_feat = jnp.ones((N_NODES, D), jnp.float32)

  sc_agg = _make_sc_agg(n_chunks)
  tc1 = _make_tc_dense(relu=True)
  tc2 = _make_tc_dense(relu=False)

  dd = sc_agg(ones_feat, src, dst)
  aa = sc_agg(x, src, dst)
  h = tc1(aa[0], aa[1], dd[0], dd[1], x, W1_l, W1_r, b1.reshape(1, D))
  cc = sc_agg(h, src, dst)
  out = tc2(cc[0], cc[1], dd[0], dd[1], h, W2_l, W2_r, b2.reshape(1, D))
  return out


# speedup vs baseline: 1.0883x; 1.0883x over previous
import functools

import jax
import jax.numpy as jnp
from jax import lax
from jax.experimental import pallas as pl
from jax.experimental.pallas import tpu as pltpu
from jax.experimental.pallas import tpu_sc as plsc

N_NODES = 10000
D = 128

NC = 2
NS = 16
NW = NC * NS

CHUNK = 128
ROWS_PER_TILE = 640
N_PAD = NS * ROWS_PER_TILE


NBUF = 2
G = 8


def _sc_agg_body(feat, idx_hbm, agg_out,
                 idx_v, rows_v,
                 ise0, ise1, sg0, sg1, agg_s):
  c = lax.axis_index("c")
  s = lax.axis_index("s")
  w = c * NS + s
  n_groups = idx_hbm.shape[1]
  sg = [sg0, sg1]
  ise = [ise0, ise1]

  pltpu.sync_copy(idx_hbm.at[w, 0], idx_v.at[0])
  if n_groups > 1:
    pltpu.async_copy(idx_hbm.at[w, 1], idx_v.at[1], ise[1])
  def zrow(i, carry):
    for j in range(D // 16):
      rows_v[0, i, pl.ds(j * 16, 16)] = jnp.zeros((16,), jnp.float32)
    return carry
  lax.fori_loop(0, CHUNK, zrow, 0)
  for zb in range(ROWS_PER_TILE // CHUNK):
    pltpu.sync_copy(rows_v.at[0],
                    agg_s.at[pl.ds(s * ROWS_PER_TILE + zb * CHUNK, CHUNK)])
  plsc.subcore_barrier()

  for b in range(NBUF):
    pltpu.async_copy(feat.at[idx_v.at[0, 0, b]], rows_v.at[b], sg[b])

  def pair_body(p, carry):
    for slot in range(2):
      nslot = 1 - slot
      g = 2 * p + slot
      for k in range(G):
        b = k % NBUF
        pltpu.make_async_copy(feat.at[idx_v.at[slot, 0, k]],
                              rows_v.at[b], sg[b]).wait()
        pltpu.sync_copy(rows_v.at[b], agg_s.at[idx_v.at[slot, 1, k]],
                        add=True)

        if k < G - NBUF:
          pltpu.async_copy(feat.at[idx_v.at[slot, 0, k + NBUF]],
                           rows_v.at[b], sg[b])
        else:
          if k == G - NBUF:
            @pl.when(g + 1 < n_groups)
            def _():
              pltpu.make_async_copy(idx_hbm.at[w, 0], idx_v.at[nslot],
                                    ise[nslot]).wait()

          @pl.when(g + 1 < n_groups)
          def _():
            pltpu.async_copy(feat.at[idx_v.at[nslot, 0, k + NBUF - G]],
                             rows_v.at[b], sg[b])
          if k == G - 1:
            @pl.when(g + 2 < n_groups)
            def _():
              pltpu.async_copy(idx_hbm.at[w, g + 2], idx_v.at[slot],
                               ise[slot])
    return carry

  lax.fori_loop(0, n_groups // 2, pair_body, 0)
  plsc.subcore_barrier()

  for blk in range(ROWS_PER_TILE // CHUNK):
    r0 = s * ROWS_PER_TILE + blk * CHUNK
    pltpu.sync_copy(agg_s.at[pl.ds(r0, CHUNK)], rows_v.at[0])
    pltpu.sync_copy(rows_v.at[0], agg_out.at[c, pl.ds(r0, CHUNK)])


def _make_sc_agg():
  mesh = plsc.VectorSubcoreMesh(core_axis_name="c", subcore_axis_name="s")
  out_type = jax.ShapeDtypeStruct((NC, N_PAD, D), jnp.float32)
  scratch = [
      pltpu.VMEM((2, 2, G, CHUNK), jnp.int32),
      pltpu.VMEM((NBUF, CHUNK, D), jnp.float32),
  ] + [pltpu.SemaphoreType.DMA] * (2 + NBUF) + [
      pltpu.VMEM_SHARED((N_PAD, D), jnp.float32),
  ]
  return pl.kernel(_sc_agg_body, out_type=out_type, mesh=mesh,
                   scratch_types=scratch, name="sc_agg")


def _tc_dense_body(relu, a0, a1, d0, d1, xr, wl, wr, b, o):
  deg = d0[:, 0:1] + d1[:, 0:1]
  inv = 1.0 / jnp.maximum(deg, 1.0)
  mean = (a0[...] + a1[...]) * inv
  acc = (jnp.dot(mean, wl[...], preferred_element_type=jnp.float32)
         + jnp.dot(xr[...], wr[...], preferred_element_type=jnp.float32)
         + b[...])
  o[...] = jnp.maximum(acc, 0.0) if relu else acc


def _make_tc_dense(relu, bn=1000):
  grid = (N_NODES // bn,)
  return pl.pallas_call(
      functools.partial(_tc_dense_body, relu),
      grid=grid,
      in_specs=[
          pl.BlockSpec((bn, D), lambda i: (i, 0)),
          pl.BlockSpec((bn, D), lambda i: (i, 0)),
          pl.BlockSpec((bn, D), lambda i: (i, 0)),
          pl.BlockSpec((bn, D), lambda i: (i, 0)),
          pl.BlockSpec((bn, D), lambda i: (i, 0)),
          pl.BlockSpec((D, D), lambda i: (0, 0)),
          pl.BlockSpec((D, D), lambda i: (0, 0)),
          pl.BlockSpec((1, D), lambda i: (0, 0)),
      ],
      out_specs=pl.BlockSpec((bn, D), lambda i: (i, 0)),
      out_shape=jax.ShapeDtypeStruct((N_NODES, D), jnp.float32),
      name="tc_dense_relu" if relu else "tc_dense",
  )


def kernel(x, edge_index, W1_l, W1_r, b1, W2_l, W2_r, b2):
  e = edge_index.shape[1]
  n_chunks = -(-e // (NW * CHUNK * 16)) * 16
  per_tile = n_chunks * CHUNK
  e_pad = NW * per_tile

  n_groups = n_chunks // G
  src = edge_index[0].astype(jnp.int32)
  dst = edge_index[1].astype(jnp.int32)
  pad = e_pad - e
  src = jnp.concatenate([src, jnp.zeros((pad,), jnp.int32)])
  dst = jnp.concatenate([dst, jnp.full((pad,), N_NODES, jnp.int32)])
  idx = jnp.stack([src.reshape(NW, n_groups, G, CHUNK),
                   dst.reshape(NW, n_groups, G, CHUNK)], axis=2)

  ones_feat = jnp.ones((N_NODES, D), jnp.float32)

  sc_agg = _make_sc_agg()
  tc1 = _make_tc_dense(relu=True)
  tc2 = _make_tc_dense(relu=False)

  dd = sc_agg(ones_feat, idx)
  aa = sc_agg(x, idx)
  h = tc1(aa[0], aa[1], dd[0], dd[1], x, W1_l, W1_r, b1.reshape(1, D))
  cc = sc_agg(h, idx)
  out = tc2(cc[0], cc[1], dd[0], dd[1], h, W2_l, W2_r, b2.reshape(1, D))
  return out


# speedup vs baseline: 1.4378x; 1.3212x over previous
import functools

import jax
import jax.numpy as jnp
from jax import lax
from jax.experimental import pallas as pl
from jax.experimental.pallas import tpu as pltpu
from jax.experimental.pallas import tpu_sc as plsc

N_NODES = 10000
D = 128

NC = 2
NS = 16
NW = NC * NS

CHUNK = 128
ROWS_PER_TILE = 640
N_PAD = NS * ROWS_PER_TILE


NBUF = 2
G = 8


def _sc_agg_body(feat, idx_hbm, agg_out,
                 idx_v, rows_v,
                 ise0, ise1, sg0, sg1, agg_s):
  c = lax.axis_index("c")
  s = lax.axis_index("s")
  w = c * NS + s
  n_groups = idx_hbm.shape[1]
  sg = [sg0, sg1]
  ise = [ise0, ise1]

  pltpu.sync_copy(idx_hbm.at[w, 0], idx_v.at[0])
  if n_groups > 1:
    pltpu.async_copy(idx_hbm.at[w, 1], idx_v.at[1], ise[1])
  def zrow(i, carry):
    for j in range(D // 16):
      rows_v[0, i, pl.ds(j * 16, 16)] = jnp.zeros((16,), jnp.float32)
    return carry
  lax.fori_loop(0, CHUNK, zrow, 0)
  for zb in range(ROWS_PER_TILE // CHUNK):
    pltpu.sync_copy(rows_v.at[0],
                    agg_s.at[pl.ds(s * ROWS_PER_TILE + zb * CHUNK, CHUNK)])
  plsc.subcore_barrier()

  for b in range(NBUF):
    pltpu.async_copy(feat.at[idx_v.at[0, 0, b]], rows_v.at[b], sg[b])

  def pair_body(p, carry):
    for slot in range(2):
      nslot = 1 - slot
      g = 2 * p + slot
      for k in range(G):
        b = k % NBUF
        pltpu.make_async_copy(feat.at[idx_v.at[slot, 0, k]],
                              rows_v.at[b], sg[b]).wait()
        pltpu.sync_copy(rows_v.at[b], agg_s.at[idx_v.at[slot, 1, k]],
                        add=True)

        if k < G - NBUF:
          pltpu.async_copy(feat.at[idx_v.at[slot, 0, k + NBUF]],
                           rows_v.at[b], sg[b])
        else:
          if k == G - NBUF:
            @pl.when(g + 1 < n_groups)
            def _():
              pltpu.make_async_copy(idx_hbm.at[w, 0], idx_v.at[nslot],
                                    ise[nslot]).wait()

          @pl.when(g + 1 < n_groups)
          def _():
            pltpu.async_copy(feat.at[idx_v.at[nslot, 0, k + NBUF - G]],
                             rows_v.at[b], sg[b])
          if k == G - 1:
            @pl.when(g + 2 < n_groups)
            def _():
              pltpu.async_copy(idx_hbm.at[w, g + 2], idx_v.at[slot],
                               ise[slot])
    return carry

  lax.fori_loop(0, n_groups // 2, pair_body, 0)
  plsc.subcore_barrier()

  for blk in range(ROWS_PER_TILE // CHUNK):
    r0 = s * ROWS_PER_TILE + blk * CHUNK
    pltpu.sync_copy(agg_s.at[pl.ds(r0, CHUNK)], rows_v.at[0])
    pltpu.sync_copy(rows_v.at[0], agg_out.at[c, pl.ds(r0, CHUNK)])


def _sc_deg_body(idx_hbm, deg_out, idx_v, rows_v, ise0, ise1, agg_s):
  c = lax.axis_index("c")
  s = lax.axis_index("s")
  w = c * NS + s
  n_groups = idx_hbm.shape[1]
  ise = [ise0, ise1]

  pltpu.sync_copy(idx_hbm.at[w, 0], idx_v.at[0])
  pltpu.async_copy(idx_hbm.at[w, 1], idx_v.at[1], ise[1])
  def fill(i, carry):
    for j in range(D // 16):
      rows_v[0, i, pl.ds(j * 16, 16)] = jnp.zeros((16,), jnp.float32)
      rows_v[1, i, pl.ds(j * 16, 16)] = jnp.ones((16,), jnp.float32)
    return carry
  lax.fori_loop(0, CHUNK, fill, 0)
  for zb in range(ROWS_PER_TILE // CHUNK):
    pltpu.sync_copy(rows_v.at[0],
                    agg_s.at[pl.ds(s * ROWS_PER_TILE + zb * CHUNK, CHUNK)])
  plsc.subcore_barrier()

  def pair_body(p, carry):
    for slot in range(2):
      g = 2 * p + slot
      if slot == 0:
        @pl.when(g > 0)
        def _():
          pltpu.make_async_copy(idx_hbm.at[w, 0], idx_v.at[0], ise[0]).wait()
      else:
        pltpu.make_async_copy(idx_hbm.at[w, 0], idx_v.at[1], ise[1]).wait()
      for k in range(G):
        pltpu.sync_copy(rows_v.at[1], agg_s.at[idx_v.at[slot, 1, k]],
                        add=True)
      @pl.when(g + 2 < n_groups)
      def _():
        pltpu.async_copy(idx_hbm.at[w, g + 2], idx_v.at[slot], ise[slot])
    return carry

  lax.fori_loop(0, n_groups // 2, pair_body, 0)
  plsc.subcore_barrier()

  for blk in range(ROWS_PER_TILE // CHUNK):
    r0 = s * ROWS_PER_TILE + blk * CHUNK
    pltpu.sync_copy(agg_s.at[pl.ds(r0, CHUNK)], rows_v.at[0])
    pltpu.sync_copy(rows_v.at[0], deg_out.at[c, pl.ds(r0, CHUNK)])


def _make_sc_deg():
  mesh = plsc.VectorSubcoreMesh(core_axis_name="c", subcore_axis_name="s")
  out_type = jax.ShapeDtypeStruct((NC, N_PAD, D), jnp.float32)
  scratch = [
      pltpu.VMEM((2, 2, G, CHUNK), jnp.int32),
      pltpu.VMEM((2, CHUNK, D), jnp.float32),
  ] + [pltpu.SemaphoreType.DMA] * 2 + [
      pltpu.VMEM_SHARED((N_PAD, D), jnp.float32),
  ]
  return pl.kernel(_sc_deg_body, out_type=out_type, mesh=mesh,
                   scratch_types=scratch, name="sc_deg")


def _make_sc_agg():
  mesh = plsc.VectorSubcoreMesh(core_axis_name="c", subcore_axis_name="s")
  out_type = jax.ShapeDtypeStruct((NC, N_PAD, D), jnp.float32)
  scratch = [
      pltpu.VMEM((2, 2, G, CHUNK), jnp.int32),
      pltpu.VMEM((NBUF, CHUNK, D), jnp.float32),
  ] + [pltpu.SemaphoreType.DMA] * (2 + NBUF) + [
      pltpu.VMEM_SHARED((N_PAD, D), jnp.float32),
  ]
  return pl.kernel(_sc_agg_body, out_type=out_type, mesh=mesh,
                   scratch_types=scratch, name="sc_agg")


def _tc_dense_body(relu, a0, a1, d0, d1, xr, wl, wr, b, o):
  deg = d0[:, 0:1] + d1[:, 0:1]
  inv = 1.0 / jnp.maximum(deg, 1.0)
  mean = (a0[...] + a1[...]) * inv
  acc = (jnp.dot(mean, wl[...], preferred_element_type=jnp.float32)
         + jnp.dot(xr[...], wr[...], preferred_element_type=jnp.float32)
         + b[...])
  o[...] = jnp.maximum(acc, 0.0) if relu else acc


def _make_tc_dense(relu, bn=1000):
  grid = (N_NODES // bn,)
  return pl.pallas_call(
      functools.partial(_tc_dense_body, relu),
      grid=grid,
      in_specs=[
          pl.BlockSpec((bn, D), lambda i: (i, 0)),
          pl.BlockSpec((bn, D), lambda i: (i, 0)),
          pl.BlockSpec((bn, D), lambda i: (i, 0)),
          pl.BlockSpec((bn, D), lambda i: (i, 0)),
          pl.BlockSpec((bn, D), lambda i: (i, 0)),
          pl.BlockSpec((D, D), lambda i: (0, 0)),
          pl.BlockSpec((D, D), lambda i: (0, 0)),
          pl.BlockSpec((1, D), lambda i: (0, 0)),
      ],
      out_specs=pl.BlockSpec((bn, D), lambda i: (i, 0)),
      out_shape=jax.ShapeDtypeStruct((N_NODES, D), jnp.float32),
      name="tc_dense_relu" if relu else "tc_dense",
  )


def kernel(x, edge_index, W1_l, W1_r, b1, W2_l, W2_r, b2):
  e = edge_index.shape[1]
  n_chunks = -(-e // (NW * CHUNK * 16)) * 16
  per_tile = n_chunks * CHUNK
  e_pad = NW * per_tile

  n_groups = n_chunks // G
  src = edge_index[0].astype(jnp.int32)
  dst = edge_index[1].astype(jnp.int32)
  pad = e_pad - e
  src = jnp.concatenate([src, jnp.zeros((pad,), jnp.int32)])
  dst = jnp.concatenate([dst, jnp.full((pad,), N_NODES, jnp.int32)])
  idx = jnp.stack([src.reshape(NW, n_groups, G, CHUNK),
                   dst.reshape(NW, n_groups, G, CHUNK)], axis=2)

  sc_agg = _make_sc_agg()
  sc_deg = _make_sc_deg()
  tc1 = _make_tc_dense(relu=True)
  tc2 = _make_tc_dense(relu=False)

  dd = sc_deg(idx)
  aa = sc_agg(x, idx)
  h = tc1(aa[0], aa[1], dd[0], dd[1], x, W1_l, W1_r, b1.reshape(1, D))
  cc = sc_agg(h, idx)
  out = tc2(cc[0], cc[1], dd[0], dd[1], h, W2_l, W2_r, b2.reshape(1, D))
  return out
